# Initial kernel scaffold; baseline (speedup 1.0000x reference)
import jax
import jax.numpy as jnp
from jax.experimental import pallas as pl


def kernel(x, edge_index, edges, edges_neg, common_neighbors, W1, b1, W2, b2):
    raise NotImplementedError("write your pallas kernel here")


# speedup vs baseline: 6.5281x; 6.5281x over previous
import functools

import jax
import jax.numpy as jnp
from jax import lax
from jax.experimental import pallas as pl
from jax.experimental.pallas import tpu as pltpu
from jax.experimental.pallas import tpu_sc as plsc

N = 10000
D = 128
E = 320000
NEV = 200000

NC = 2
NS = 16
NW = NC * NS

CH = 128
E_CHUNKS = E // CH
NH = 10240
H_PER_TILE = NH // NS
ZB = 80
NEV_PAD = 200064
EV_CHUNKS = NEV_PAD // CH

_mesh = plsc.VectorSubcoreMesh(core_axis_name="c", subcore_axis_name="s")

_Z16 = functools.partial(jnp.zeros, (16,), jnp.float32)


def _wid():
    return lax.axis_index("s") * NC + lax.axis_index("c")


@functools.partial(
    pl.kernel,
    out_type=jax.ShapeDtypeStruct((NC * NH,), jnp.float32),
    mesh=_mesh,
    scratch_types=[
        pltpu.VMEM((CH,), jnp.int32),
        pltpu.VMEM((CH,), jnp.float32),
        pltpu.VMEM((H_PER_TILE,), jnp.float32),
        pltpu.VMEM_SHARED((NH,), jnp.float32),
    ],
)
def _sc_degree(dst_hbm, out_hbm, idx_v, ones_v, zeros_v, hist):
    cid = lax.axis_index("c")
    sid = lax.axis_index("s")
    wid = _wid()

    def fill(i, _):
        ones_v[pl.ds(i * 16, 16)] = jnp.full((16,), 1.0, jnp.float32)
        return 0
    lax.fori_loop(0, CH // 16, fill, 0)

    def zfill(i, _):
        zeros_v[pl.ds(i * 16, 16)] = _Z16()
        return 0
    lax.fori_loop(0, H_PER_TILE // 16, zfill, 0)

    pltpu.sync_copy(zeros_v, hist.at[pl.ds(sid * H_PER_TILE, H_PER_TILE)])
    plsc.subcore_barrier()

    nchunks = (E_CHUNKS - wid + NW - 1) // NW

    def body(i, _):
        off = (wid + i * NW) * CH
        pltpu.sync_copy(dst_hbm.at[pl.ds(off, CH)], idx_v)
        pltpu.sync_copy(ones_v, hist.at[idx_v], add=True)
        return 0
    lax.fori_loop(0, nchunks, body, 0)

    plsc.subcore_barrier()
    pltpu.sync_copy(
        hist.at[pl.ds(sid * H_PER_TILE, H_PER_TILE)],
        out_hbm.at[pl.ds(cid * NH + sid * H_PER_TILE, H_PER_TILE)],
    )


def _rsqrt16(x):
    i = lax.bitcast_convert_type(x, jnp.int32)
    i = jnp.int32(0x5F3759DF) - lax.shift_right_arithmetic(i, 1)
    y = lax.bitcast_convert_type(i, jnp.float32)
    for _ in range(3):
        y = y * (1.5 - 0.5 * x * y * y)
    return y


DINV_PER_TILE = NH // NW


@functools.partial(
    pl.kernel,
    out_type=jax.ShapeDtypeStruct((NH, D), jnp.float32),
    mesh=_mesh,
    scratch_types=[
        pltpu.VMEM((DINV_PER_TILE,), jnp.float32),
        pltpu.VMEM((DINV_PER_TILE,), jnp.float32),
        pltpu.VMEM((DINV_PER_TILE, D), jnp.float32),
    ],
)
def _sc_dinv(cnt_hbm, out_hbm, c0_v, c1_v, rows_v):
    wid = _wid()
    base = wid * DINV_PER_TILE
    pltpu.sync_copy(cnt_hbm.at[pl.ds(base, DINV_PER_TILE)], c0_v)
    pltpu.sync_copy(cnt_hbm.at[pl.ds(NH + base, DINV_PER_TILE)], c1_v)

    def body(g, _):
        deg = c0_v[pl.ds(g * 16, 16)] + c1_v[pl.ds(g * 16, 16)] + 1.0
        y = _rsqrt16(deg)
        for j in range(16):
            row = jnp.broadcast_to(y[j], (16,))
            for k in range(D // 16):
                rows_v[g * 16 + j, pl.ds(k * 16, 16)] = row
        return 0
    lax.fori_loop(0, DINV_PER_TILE // 16, body, 0)

    pltpu.sync_copy(rows_v, out_hbm.at[pl.ds(base, DINV_PER_TILE)])


@functools.partial(
    pl.kernel,
    out_type=jax.ShapeDtypeStruct((NC * N, D), jnp.float32),
    mesh=_mesh,
    scratch_types=[
        pltpu.VMEM((CH,), jnp.int32),
        pltpu.VMEM((CH,), jnp.int32),
        pltpu.VMEM((CH, D), jnp.float32),
        pltpu.VMEM_SHARED((N, D), jnp.float32),
        pltpu.SemaphoreType.DMA,
    ],
)
def _sc_scatter(g_hbm, src_hbm, dst_hbm, out_hbm, src_v, dst_v, rows_v, acc, sem):
    cid = lax.axis_index("c")
    sid = lax.axis_index("s")
    wid = _wid()

    def zrow(j, _):
        for k in range(D // 16):
            rows_v[j, pl.ds(k * 16, 16)] = _Z16()
        return 0
    lax.fori_loop(0, CH, zrow, 0)

    nzb = (N // ZB - sid + NS - 1) // NS

    def zblk(i, _):
        b = sid + i * NS
        pltpu.sync_copy(rows_v.at[pl.ds(0, ZB)], acc.at[pl.ds(b * ZB, ZB)])
        return 0
    lax.fori_loop(0, nzb, zblk, 0)
    plsc.subcore_barrier()

    nchunks = (E_CHUNKS - wid + NW - 1) // NW

    def body(i, _):
        off = (wid + i * NW) * CH
        pltpu.sync_copy(src_hbm.at[pl.ds(off, CH)], src_v)
        pltpu.sync_copy(dst_hbm.at[pl.ds(off, CH)], dst_v)
        pltpu.async_copy(g_hbm.at[src_v], rows_v, sem).wait()
        pltpu.sync_copy(rows_v, acc.at[dst_v], add=True)
        return 0
    lax.fori_loop(0, nchunks, body, 0)

    plsc.subcore_barrier()

    def oblk(i, _):
        b = sid + i * NS
        pltpu.sync_copy(acc.at[pl.ds(b * ZB, ZB)],
                        out_hbm.at[pl.ds(cid * N + b * ZB, ZB)])
        return 0
    lax.fori_loop(0, nzb, oblk, 0)


@functools.partial(
    pl.kernel,
    out_type=jax.ShapeDtypeStruct((NEV_PAD,), jnp.float32),
    mesh=_mesh,
    scratch_types=[
        pltpu.VMEM((CH,), jnp.int32),
        pltpu.VMEM((CH,), jnp.int32),
        pltpu.VMEM((CH, D), jnp.float32),
        pltpu.VMEM((CH, D), jnp.float32),
        pltpu.VMEM((CH,), jnp.float32),
        pltpu.SemaphoreType.DMA,
        pltpu.SemaphoreType.DMA,
    ],
    compiler_params=pltpu.CompilerParams(needs_layout_passes=False),
)
def _sc_eval(h_hbm, ia_hbm, ib_hbm, out_hbm, ia_v, ib_v, a_v, b_v, o_v, sem_a, sem_b):
    wid = _wid()
    lanes = lax.iota(jnp.int32, 16)
    nchunks = (EV_CHUNKS - wid + NW - 1) // NW

    def body(i, _):
        off = (wid + i * NW) * CH
        pltpu.sync_copy(ia_hbm.at[pl.ds(off, CH)], ia_v)
        pltpu.sync_copy(ib_hbm.at[pl.ds(off, CH)], ib_v)
        ca = pltpu.async_copy(h_hbm.at[ia_v], a_v, sem_a)
        cb = pltpu.async_copy(h_hbm.at[ib_v], b_v, sem_b)
        ca.wait()
        cb.wait()

        def group(g, _):
            rows = g * 16 + lanes
            accs = [_Z16() for _ in range(4)]
            for k in range(D):
                col = jnp.full((16,), k, jnp.int32)
                accs[k % 4] = accs[k % 4] + (
                    plsc.load_gather(a_v, [rows, col])
                    * plsc.load_gather(b_v, [rows, col])
                )
            o_v[pl.ds(g * 16, 16)] = (accs[0] + accs[1]) + (accs[2] + accs[3])
            return 0
        lax.fori_loop(0, CH // 16, group, 0)
        pltpu.sync_copy(o_v, out_hbm.at[pl.ds(off, CH)])
        return 0
    lax.fori_loop(0, nchunks, body, 0)


RB = 2000
_GRID = N // RB


def _tc_in_body(x_ref, w_ref, dv_ref, g_ref):
    h = jnp.dot(x_ref[...], w_ref[...], preferred_element_type=jnp.float32)
    g_ref[...] = h * dv_ref[...]


_tc_in = pl.pallas_call(
    _tc_in_body,
    grid=(_GRID,),
    in_specs=[
        pl.BlockSpec((RB, D), lambda i: (i, 0)),
        pl.BlockSpec((D, D), lambda i: (0, 0)),
        pl.BlockSpec((RB, D), lambda i: (i, 0)),
    ],
    out_specs=pl.BlockSpec((RB, D), lambda i: (i, 0)),
    out_shape=jax.ShapeDtypeStruct((N, D), jnp.float32),
)


def _tc_mid_body(p_ref, g1_ref, dv_ref, w_ref, b_ref, g2_ref):
    dv = dv_ref[...]
    s = (p_ref[0] + p_ref[1] + g1_ref[...]) * dv + b_ref[...]
    h1 = jnp.maximum(s, 0.0)
    g2_ref[...] = jnp.dot(h1, w_ref[...], preferred_element_type=jnp.float32) * dv


_tc_mid = pl.pallas_call(
    _tc_mid_body,
    grid=(_GRID,),
    in_specs=[
        pl.BlockSpec((2, RB, D), lambda i: (0, i, 0)),
        pl.BlockSpec((RB, D), lambda i: (i, 0)),
        pl.BlockSpec((RB, D), lambda i: (i, 0)),
        pl.BlockSpec((D, D), lambda i: (0, 0)),
        pl.BlockSpec((1, D), lambda i: (0, 0)),
    ],
    out_specs=pl.BlockSpec((RB, D), lambda i: (i, 0)),
    out_shape=jax.ShapeDtypeStruct((N, D), jnp.float32),
)


def _tc_fin_body(p_ref, g2_ref, dv_ref, b_ref, h2_ref):
    h2_ref[...] = (p_ref[0] + p_ref[1] + g2_ref[...]) * dv_ref[...] + b_ref[...]


_tc_fin = pl.pallas_call(
    _tc_fin_body,
    grid=(_GRID,),
    in_specs=[
        pl.BlockSpec((2, RB, D), lambda i: (0, i, 0)),
        pl.BlockSpec((RB, D), lambda i: (i, 0)),
        pl.BlockSpec((RB, D), lambda i: (i, 0)),
        pl.BlockSpec((1, D), lambda i: (0, 0)),
    ],
    out_specs=pl.BlockSpec((RB, D), lambda i: (i, 0)),
    out_shape=jax.ShapeDtypeStruct((N, D), jnp.float32),
)


def kernel(x, edge_index, edges, edges_neg, common_neighbors, W1, b1, W2, b2):
    del common_neighbors
    ei = edge_index.astype(jnp.int32)
    src = ei[0]
    dst = ei[1]

    cnt = _sc_degree(dst)
    dinv_bc = _sc_dinv(cnt)[:N]

    g1 = _tc_in(x, W1, dinv_bc)
    parts1 = _sc_scatter(g1, src, dst).reshape(NC, N, D)
    g2 = _tc_mid(parts1, g1, dinv_bc, W2, b1.reshape(1, D))
    parts2 = _sc_scatter(g2, src, dst).reshape(NC, N, D)
    h2 = _tc_fin(parts2, g2, dinv_bc, b2.reshape(1, D))

    pad = jnp.zeros((NEV_PAD - NEV,), jnp.int32)
    ia = jnp.concatenate([edges[:, 0].astype(jnp.int32),
                          edges_neg[:, 0].astype(jnp.int32), pad])
    ib = jnp.concatenate([edges[:, 1].astype(jnp.int32),
                          edges_neg[:, 1].astype(jnp.int32), pad])
    out = _sc_eval(h2, ia, ib)
    return out[:NEV]


# speedup vs baseline: 10.9535x; 1.6779x over previous
import functools

import jax
import jax.numpy as jnp
from jax import lax
from jax.experimental import pallas as pl
from jax.experimental.pallas import tpu as pltpu
from jax.experimental.pallas import tpu_sc as plsc

N = 10000
D = 128
E = 320000
NEV = 200000

NC = 2
NS = 16
NW = NC * NS

CH = 128
E_CHUNKS = E // CH
NH = 10240
H_PER_TILE = NH // NS
ZB = 80
NEV_PAD = 200064
EV_CHUNKS = NEV_PAD // CH

_mesh = plsc.VectorSubcoreMesh(core_axis_name="c", subcore_axis_name="s")

_Z16 = functools.partial(jnp.zeros, (16,), jnp.float32)


def _wid():
    return lax.axis_index("s") * NC + lax.axis_index("c")


@functools.partial(
    pl.kernel,
    out_type=jax.ShapeDtypeStruct((NC * NH,), jnp.float32),
    mesh=_mesh,
    scratch_types=[
        pltpu.VMEM((CH,), jnp.int32),
        pltpu.VMEM((CH,), jnp.float32),
        pltpu.VMEM((H_PER_TILE,), jnp.float32),
        pltpu.VMEM_SHARED((NH,), jnp.float32),
    ],
)
def _sc_degree(dst_hbm, out_hbm, idx_v, ones_v, zeros_v, hist):
    cid = lax.axis_index("c")
    sid = lax.axis_index("s")
    wid = _wid()

    def fill(i, _):
        ones_v[pl.ds(i * 16, 16)] = jnp.full((16,), 1.0, jnp.float32)
        return 0
    lax.fori_loop(0, CH // 16, fill, 0)

    def zfill(i, _):
        zeros_v[pl.ds(i * 16, 16)] = _Z16()
        return 0
    lax.fori_loop(0, H_PER_TILE // 16, zfill, 0)

    pltpu.sync_copy(zeros_v, hist.at[pl.ds(sid * H_PER_TILE, H_PER_TILE)])
    plsc.subcore_barrier()

    nchunks = (E_CHUNKS - wid + NW - 1) // NW

    def body(i, _):
        off = (wid + i * NW) * CH
        pltpu.sync_copy(dst_hbm.at[pl.ds(off, CH)], idx_v)
        pltpu.sync_copy(ones_v, hist.at[idx_v], add=True)
        return 0
    lax.fori_loop(0, nchunks, body, 0)

    plsc.subcore_barrier()
    pltpu.sync_copy(
        hist.at[pl.ds(sid * H_PER_TILE, H_PER_TILE)],
        out_hbm.at[pl.ds(cid * NH + sid * H_PER_TILE, H_PER_TILE)],
    )


def _rsqrt16(x):
    i = lax.bitcast_convert_type(x, jnp.int32)
    i = jnp.int32(0x5F3759DF) - lax.shift_right_arithmetic(i, 1)
    y = lax.bitcast_convert_type(i, jnp.float32)
    for _ in range(3):
        y = y * (1.5 - 0.5 * x * y * y)
    return y


DINV_PER_TILE = NH // NW


@functools.partial(
    pl.kernel,
    out_type=jax.ShapeDtypeStruct((NH, D), jnp.float32),
    mesh=_mesh,
    scratch_types=[
        pltpu.VMEM((DINV_PER_TILE,), jnp.float32),
        pltpu.VMEM((DINV_PER_TILE,), jnp.float32),
        pltpu.VMEM((DINV_PER_TILE, D), jnp.float32),
    ],
)
def _sc_dinv(cnt_hbm, out_hbm, c0_v, c1_v, rows_v):
    wid = _wid()
    base = wid * DINV_PER_TILE
    pltpu.sync_copy(cnt_hbm.at[pl.ds(base, DINV_PER_TILE)], c0_v)
    pltpu.sync_copy(cnt_hbm.at[pl.ds(NH + base, DINV_PER_TILE)], c1_v)

    def body(g, _):
        deg = c0_v[pl.ds(g * 16, 16)] + c1_v[pl.ds(g * 16, 16)] + 1.0
        y = _rsqrt16(deg)
        for j in range(16):
            row = jnp.broadcast_to(y[j], (16,))
            for k in range(D // 16):
                rows_v[g * 16 + j, pl.ds(k * 16, 16)] = row
        return 0
    lax.fori_loop(0, DINV_PER_TILE // 16, body, 0)

    pltpu.sync_copy(rows_v, out_hbm.at[pl.ds(base, DINV_PER_TILE)])


@functools.partial(
    pl.kernel,
    out_type=jax.ShapeDtypeStruct((NC * N, D), jnp.float32),
    mesh=_mesh,
    scratch_types=[
        pltpu.VMEM((CH,), jnp.int32),
        pltpu.VMEM((CH,), jnp.int32),
        pltpu.VMEM((CH, D), jnp.float32),
        pltpu.VMEM_SHARED((N, D), jnp.float32),
        pltpu.SemaphoreType.DMA,
    ],
)
def _sc_scatter(g_hbm, src_hbm, dst_hbm, out_hbm, src_v, dst_v, rows_v, acc, sem):
    cid = lax.axis_index("c")
    sid = lax.axis_index("s")
    wid = _wid()

    def zrow(j, _):
        for k in range(D // 16):
            rows_v[j, pl.ds(k * 16, 16)] = _Z16()
        return 0
    lax.fori_loop(0, CH, zrow, 0)

    nzb = (N // ZB - sid + NS - 1) // NS

    def zblk(i, _):
        b = sid + i * NS
        pltpu.sync_copy(rows_v.at[pl.ds(0, ZB)], acc.at[pl.ds(b * ZB, ZB)])
        return 0
    lax.fori_loop(0, nzb, zblk, 0)
    plsc.subcore_barrier()

    nchunks = (E_CHUNKS - wid + NW - 1) // NW

    def body(i, _):
        off = (wid + i * NW) * CH
        pltpu.sync_copy(src_hbm.at[pl.ds(off, CH)], src_v)
        pltpu.sync_copy(dst_hbm.at[pl.ds(off, CH)], dst_v)
        pltpu.async_copy(g_hbm.at[src_v], rows_v, sem).wait()
        pltpu.sync_copy(rows_v, acc.at[dst_v], add=True)
        return 0
    lax.fori_loop(0, nchunks, body, 0)

    plsc.subcore_barrier()

    def oblk(i, _):
        b = sid + i * NS
        pltpu.sync_copy(acc.at[pl.ds(b * ZB, ZB)],
                        out_hbm.at[pl.ds(cid * N + b * ZB, ZB)])
        return 0
    lax.fori_loop(0, nzb, oblk, 0)


@functools.partial(
    pl.kernel,
    out_type=jax.ShapeDtypeStruct((NEV_PAD,), jnp.float32),
    mesh=_mesh,
    scratch_types=[
        pltpu.VMEM((CH,), jnp.int32),
        pltpu.VMEM((CH,), jnp.int32),
        pltpu.VMEM((CH, D), jnp.float32),
        pltpu.VMEM((CH, D), jnp.float32),
        pltpu.VMEM((CH,), jnp.float32),
        pltpu.SemaphoreType.DMA,
        pltpu.SemaphoreType.DMA,
    ],
    compiler_params=pltpu.CompilerParams(needs_layout_passes=False),
)
def _sc_eval(h_hbm, ia_hbm, ib_hbm, out_hbm, ia_v, ib_v, a_v, b_v, o_v, sem_a, sem_b):
    wid = _wid()
    lanes = lax.iota(jnp.int32, 16)
    nchunks = (EV_CHUNKS - wid + NW - 1) // NW

    def body(i, _):
        off = (wid + i * NW) * CH
        pltpu.sync_copy(ia_hbm.at[pl.ds(off, CH)], ia_v)
        pltpu.sync_copy(ib_hbm.at[pl.ds(off, CH)], ib_v)
        ca = pltpu.async_copy(h_hbm.at[ia_v], a_v, sem_a)
        cb = pltpu.async_copy(h_hbm.at[ib_v], b_v, sem_b)
        ca.wait()
        cb.wait()

        def group(g, _):
            ovec = _Z16()
            for j in range(16):
                p = g * 16 + j
                acc0 = a_v[p, pl.ds(0, 16)] * b_v[p, pl.ds(0, 16)]
                acc1 = a_v[p, pl.ds(16, 16)] * b_v[p, pl.ds(16, 16)]
                for k in range(2, D // 16, 2):
                    acc0 = acc0 + a_v[p, pl.ds(k * 16, 16)] * b_v[p, pl.ds(k * 16, 16)]
                    acc1 = acc1 + a_v[p, pl.ds((k + 1) * 16, 16)] * b_v[p, pl.ds((k + 1) * 16, 16)]
                s = jnp.sum(acc0 + acc1)
                ovec = jnp.where(lanes == j, s, ovec)
            o_v[pl.ds(g * 16, 16)] = ovec
            return 0
        lax.fori_loop(0, CH // 16, group, 0)
        pltpu.sync_copy(o_v, out_hbm.at[pl.ds(off, CH)])
        return 0
    lax.fori_loop(0, nchunks, body, 0)


RB = 2000
_GRID = N // RB


def _tc_in_body(x_ref, w_ref, dv_ref, g_ref):
    h = jnp.dot(x_ref[...], w_ref[...], preferred_element_type=jnp.float32)
    g_ref[...] = h * dv_ref[...]


_tc_in = pl.pallas_call(
    _tc_in_body,
    grid=(_GRID,),
    in_specs=[
        pl.BlockSpec((RB, D), lambda i: (i, 0)),
        pl.BlockSpec((D, D), lambda i: (0, 0)),
        pl.BlockSpec((RB, D), lambda i: (i, 0)),
    ],
    out_specs=pl.BlockSpec((RB, D), lambda i: (i, 0)),
    out_shape=jax.ShapeDtypeStruct((N, D), jnp.float32),
)


def _tc_mid_body(p_ref, g1_ref, dv_ref, w_ref, b_ref, g2_ref):
    dv = dv_ref[...]
    s = (p_ref[0] + p_ref[1] + g1_ref[...]) * dv + b_ref[...]
    h1 = jnp.maximum(s, 0.0)
    g2_ref[...] = jnp.dot(h1, w_ref[...], preferred_element_type=jnp.float32) * dv


_tc_mid = pl.pallas_call(
    _tc_mid_body,
    grid=(_GRID,),
    in_specs=[
        pl.BlockSpec((2, RB, D), lambda i: (0, i, 0)),
        pl.BlockSpec((RB, D), lambda i: (i, 0)),
        pl.BlockSpec((RB, D), lambda i: (i, 0)),
        pl.BlockSpec((D, D), lambda i: (0, 0)),
        pl.BlockSpec((1, D), lambda i: (0, 0)),
    ],
    out_specs=pl.BlockSpec((RB, D), lambda i: (i, 0)),
    out_shape=jax.ShapeDtypeStruct((N, D), jnp.float32),
)


def _tc_fin_body(p_ref, g2_ref, dv_ref, b_ref, h2_ref):
    h2_ref[...] = (p_ref[0] + p_ref[1] + g2_ref[...]) * dv_ref[...] + b_ref[...]


_tc_fin = pl.pallas_call(
    _tc_fin_body,
    grid=(_GRID,),
    in_specs=[
        pl.BlockSpec((2, RB, D), lambda i: (0, i, 0)),
        pl.BlockSpec((RB, D), lambda i: (i, 0)),
        pl.BlockSpec((RB, D), lambda i: (i, 0)),
        pl.BlockSpec((1, D), lambda i: (0, 0)),
    ],
    out_specs=pl.BlockSpec((RB, D), lambda i: (i, 0)),
    out_shape=jax.ShapeDtypeStruct((N, D), jnp.float32),
)


def kernel(x, edge_index, edges, edges_neg, common_neighbors, W1, b1, W2, b2):
    del common_neighbors
    ei = edge_index.astype(jnp.int32)
    src = ei[0]
    dst = ei[1]

    cnt = _sc_degree(dst)
    dinv_bc = _sc_dinv(cnt)[:N]

    g1 = _tc_in(x, W1, dinv_bc)
    parts1 = _sc_scatter(g1, src, dst).reshape(NC, N, D)
    g2 = _tc_mid(parts1, g1, dinv_bc, W2, b1.reshape(1, D))
    parts2 = _sc_scatter(g2, src, dst).reshape(NC, N, D)
    h2 = _tc_fin(parts2, g2, dinv_bc, b2.reshape(1, D))

    pad = jnp.zeros((NEV_PAD - NEV,), jnp.int32)
    ia = jnp.concatenate([edges[:, 0].astype(jnp.int32),
                          edges_neg[:, 0].astype(jnp.int32), pad])
    ib = jnp.concatenate([edges[:, 1].astype(jnp.int32),
                          edges_neg[:, 1].astype(jnp.int32), pad])
    out = _sc_eval(h2, ia, ib)
    return out[:NEV]
